# baseline (speedup 1.0000x reference)
import functools

import jax
import jax.numpy as jnp
from jax import lax
from jax.experimental import pallas as pl
from jax.experimental.pallas import tpu as pltpu
from jax.experimental.pallas import tpu_sc as plsc

N_NODES = 10000
N_EDGES = 320000
D = 128

NC = 2
NS = 16
L = 16
NW = NC * NS
K = 80
NCK0 = 162
NCK1 = 90
EPW0 = K * NCK0
EPW1 = K * NCK1
EPW_MAX = max(EPW0, EPW1)
E_PAD = NS * (EPW0 + EPW1)
N_PAD = 10240
ROWS_PER_TILE = N_PAD // NS
NR = 3


_GDN = lax.GatherDimensionNumbers(
    offset_dims=(), collapsed_slice_dims=(0,), start_index_map=(0,))


def _edge_agg(x, src, dst, ew):
    mesh = plsc.VectorSubcoreMesh(core_axis_name="c", subcore_axis_name="s",
                                  num_cores=NC, num_subcores=NS)

    @functools.partial(
        pl.kernel,
        out_type=jax.ShapeDtypeStruct((NC * N_PAD, D), jnp.float32),
        mesh=mesh,
        scratch_types=(
            [pltpu.VMEM((EPW_MAX,), jnp.int32)]
            + [pltpu.VMEM((K,), jnp.int32)] * NR
            + [pltpu.VMEM((K,), jnp.float32)] * NR
            + [pltpu.VMEM((K, D), jnp.float32)] * NR
            + [pltpu.VMEM_SHARED((N_PAD, D), jnp.float32)]
            + [pltpu.SemaphoreType.DMA] * (4 * NR)
        ),
    )
    def body(x_hbm, src_hbm, dst_hbm, ew_hbm, out_hbm,
             src_v, dst0, dst1, dst2, w0, w1s, w2s, rows0, rows1, rows2,
             acc_sh, g0, g1, g2, ws0, ws1, ws2, d0, d1, d2, s0, s1, s2):
        dst_v = [dst0, dst1, dst2]
        w_v = [w0, w1s, w2s]
        rows_v = [rows0, rows1, rows2]
        gsem = [g0, g1, g2]
        wsem = [ws0, ws1, ws2]
        dsem = [d0, d1, d2]
        ssem = [s0, s1, s2]
        cid = lax.axis_index("c")
        sid = lax.axis_index("s")
        base = jnp.where(cid == 0, sid * EPW0, NS * EPW0 + sid * EPW1)
        nck = jnp.where(cid == 0, NCK0, NCK1)
        zero16 = lax.iota(jnp.int32, L) * 0

        zv = jnp.zeros((L,), jnp.float32)

        def zrow(e, carry):
            for c in range(D // L):
                rows_v[0][e, pl.ds(c * L, L)] = zv
            return carry

        lax.fori_loop(0, K, zrow, 0)
        for j in range(ROWS_PER_TILE // K):
            pltpu.sync_copy(
                rows_v[0], acc_sh.at[pl.ds(sid * ROWS_PER_TILE + j * K, K)])
        plsc.subcore_barrier()

        pltpu.sync_copy(src_hbm.at[pl.ds(base, min(EPW0, EPW1))], src_v.at[pl.ds(0, min(EPW0, EPW1))])

        @pl.when(nck == max(NCK0, NCK1))
        def _():
            lo = min(EPW0, EPW1)
            pltpu.sync_copy(src_hbm.at[pl.ds(base + lo, EPW_MAX - lo)],
                            src_v.at[pl.ds(lo, EPW_MAX - lo)])

        def src_slice(i):
            iK = i * K
            if not isinstance(iK, int):
                iK = pl.multiple_of(iK, K)
            return src_v.at[pl.ds(iK, K)]

        def start_chunk(i, r):
            off = base + i * K
            pltpu.async_copy(dst_hbm.at[pl.ds(off, K)], dst_v[r], dsem[r])
            pltpu.async_copy(ew_hbm.at[pl.ds(off, K)], w_v[r], wsem[r])
            pltpu.async_copy(x_hbm.at[src_slice(i)], rows_v[r], gsem[r])

        def wait_chunk(i, r):
            off = base + i * K
            pltpu.make_async_copy(ew_hbm.at[pl.ds(off, K)],
                                  w_v[r], wsem[r]).wait()
            pltpu.make_async_copy(dst_hbm.at[pl.ds(off, K)],
                                  dst_v[r], dsem[r]).wait()
            pltpu.make_async_copy(x_hbm.at[src_slice(i)],
                                  rows_v[r], gsem[r]).wait()

        def wait_scatter(r):
            pltpu.make_async_copy(rows_v[r], acc_sh.at[dst_v[r]],
                                  ssem[r]).wait()

        def chunk_body(i, u, first=False, last=False):
            r = u % NR
            wait_chunk(i, r)
            def scale(g, c2):
                w16 = w_v[r][pl.ds(pl.multiple_of(g * L, L), L)]
                for e16 in range(L):
                    wsp = lax.gather(
                        w16, (zero16 + e16)[:, None], dimension_numbers=_GDN,
                        slice_sizes=(1,),
                        mode=lax.GatherScatterMode.PROMISE_IN_BOUNDS)
                    eg = g * L + e16
                    for c in range(D // L):
                        csl = pl.ds(c * L, L)
                        rows_v[r][eg, csl] = rows_v[r][eg, csl] * wsp
                return c2

            lax.fori_loop(0, K // L, scale, 0)
            pltpu.async_copy(rows_v[r], acc_sh.at[dst_v[r]],
                             ssem[r], add=True)
            if not first:
                wait_scatter((r + 2) % NR)
            if not last:
                start_chunk(i + 2, (r + 2) % NR)

        for u in range(2):
            start_chunk(u, u)
        for u in range(3):
            chunk_body(u, u, first=(u == 0))

        def group(p, carry):
            for u in range(3):
                chunk_body(3 * p + u, u)
            return carry

        lax.fori_loop(1, nck // 3 - 1, group, 0)
        for u in range(3):
            i = nck - 3 + u
            chunk_body(i, u, last=(u >= 1))
        wait_scatter(2)
        plsc.subcore_barrier()

        for j in range(ROWS_PER_TILE // K):
            row0 = sid * ROWS_PER_TILE + j * K
            pltpu.sync_copy(acc_sh.at[pl.ds(row0, K)], rows_v[0])
            pltpu.sync_copy(
                rows_v[0], out_hbm.at[pl.ds(cid * N_PAD + row0, K)])

    return body(x, src, dst, ew)


def _finish(partials, x, w1, w2, b):
    def body(p_ref, x_ref, w1_ref, w2_ref, b_ref, o_ref):
        g = p_ref[0] + p_ref[1]
        acc = jnp.dot(x_ref[...], w1_ref[...],
                      preferred_element_type=jnp.float32)
        acc += jnp.dot(g, w2_ref[...], preferred_element_type=jnp.float32)
        o_ref[...] = jnp.maximum(acc + b_ref[...], 0.0)

    grid = 10
    blk = N_NODES // grid
    return pl.pallas_call(
        body,
        grid=(grid,),
        in_specs=[
            pl.BlockSpec((NC, blk, D), lambda i: (0, i, 0)),
            pl.BlockSpec((blk, D), lambda i: (i, 0)),
            pl.BlockSpec((D, D), lambda i: (0, 0)),
            pl.BlockSpec((D, D), lambda i: (0, 0)),
            pl.BlockSpec((D,), lambda i: (0,)),
        ],
        out_specs=pl.BlockSpec((blk, D), lambda i: (i, 0)),
        out_shape=jax.ShapeDtypeStruct((N_NODES, D), jnp.float32),
    )(partials, x, w1, w2, b)


@jax.jit
def kernel(X, edge_index, edge_weight, w1, w2, b):
    src = edge_index[1].astype(jnp.int32)
    dst = edge_index[0].astype(jnp.int32)
    pad = E_PAD - N_EDGES
    src_p = jnp.concatenate([src, jnp.zeros((pad,), jnp.int32)])
    dst_p = jnp.concatenate([dst, jnp.zeros((pad,), jnp.int32)])
    ew_p = jnp.concatenate([edge_weight, jnp.zeros((pad,), jnp.float32)])
    p = _edge_agg(X, src_p, dst_p, ew_p)
    return _finish(p.reshape(NC, N_PAD, D), X, w1, w2, b)


# speedup vs baseline: 1.0588x; 1.0588x over previous
import functools

import jax
import jax.numpy as jnp
from jax import lax
from jax.experimental import pallas as pl
from jax.experimental.pallas import tpu as pltpu
from jax.experimental.pallas import tpu_sc as plsc

N_NODES = 10000
N_EDGES = 320000
D = 128

NC = 2
NS = 16
L = 16
NW = NC * NS
K = 80
NCK0 = 180
NCK1 = 72
EPW0 = K * NCK0
EPW1 = K * NCK1
EPW_MAX = max(EPW0, EPW1)
E_PAD = NS * (EPW0 + EPW1)
N_PAD = 10240
ROWS_PER_TILE = N_PAD // NS
NR = 3


_GDN = lax.GatherDimensionNumbers(
    offset_dims=(), collapsed_slice_dims=(0,), start_index_map=(0,))


def _edge_agg(x, src, dst, ew):
    mesh = plsc.VectorSubcoreMesh(core_axis_name="c", subcore_axis_name="s",
                                  num_cores=NC, num_subcores=NS)

    @functools.partial(
        pl.kernel,
        out_type=jax.ShapeDtypeStruct((NC * N_PAD, D), jnp.float32),
        mesh=mesh,
        scratch_types=(
            [pltpu.VMEM((EPW_MAX,), jnp.int32)]
            + [pltpu.VMEM((K,), jnp.int32)] * NR
            + [pltpu.VMEM((K,), jnp.float32)] * NR
            + [pltpu.VMEM((K, D), jnp.float32)] * NR
            + [pltpu.VMEM_SHARED((N_PAD, D), jnp.float32)]
            + [pltpu.SemaphoreType.DMA] * (4 * NR)
        ),
    )
    def body(x_hbm, src_hbm, dst_hbm, ew_hbm, out_hbm,
             src_v, dst0, dst1, dst2, w0, w1s, w2s, rows0, rows1, rows2,
             acc_sh, g0, g1, g2, ws0, ws1, ws2, d0, d1, d2, s0, s1, s2):
        dst_v = [dst0, dst1, dst2]
        w_v = [w0, w1s, w2s]
        rows_v = [rows0, rows1, rows2]
        gsem = [g0, g1, g2]
        wsem = [ws0, ws1, ws2]
        dsem = [d0, d1, d2]
        ssem = [s0, s1, s2]
        cid = lax.axis_index("c")
        sid = lax.axis_index("s")
        base = jnp.where(cid == 0, sid * EPW0, NS * EPW0 + sid * EPW1)
        nck = jnp.where(cid == 0, NCK0, NCK1)
        zero16 = lax.iota(jnp.int32, L) * 0

        zv = jnp.zeros((L,), jnp.float32)

        def zrow(e, carry):
            for c in range(D // L):
                rows_v[0][e, pl.ds(c * L, L)] = zv
            return carry

        lax.fori_loop(0, K, zrow, 0)
        for j in range(ROWS_PER_TILE // K):
            pltpu.sync_copy(
                rows_v[0], acc_sh.at[pl.ds(sid * ROWS_PER_TILE + j * K, K)])
        plsc.subcore_barrier()

        pltpu.sync_copy(src_hbm.at[pl.ds(base, min(EPW0, EPW1))], src_v.at[pl.ds(0, min(EPW0, EPW1))])

        @pl.when(nck == max(NCK0, NCK1))
        def _():
            lo = min(EPW0, EPW1)
            pltpu.sync_copy(src_hbm.at[pl.ds(base + lo, EPW_MAX - lo)],
                            src_v.at[pl.ds(lo, EPW_MAX - lo)])

        def src_slice(i):
            iK = i * K
            if not isinstance(iK, int):
                iK = pl.multiple_of(iK, K)
            return src_v.at[pl.ds(iK, K)]

        def start_chunk(i, r):
            off = base + i * K
            pltpu.async_copy(dst_hbm.at[pl.ds(off, K)], dst_v[r], dsem[r])
            pltpu.async_copy(ew_hbm.at[pl.ds(off, K)], w_v[r], wsem[r])
            pltpu.async_copy(x_hbm.at[src_slice(i)], rows_v[r], gsem[r])

        def wait_chunk(i, r):
            off = base + i * K
            pltpu.make_async_copy(ew_hbm.at[pl.ds(off, K)],
                                  w_v[r], wsem[r]).wait()
            pltpu.make_async_copy(dst_hbm.at[pl.ds(off, K)],
                                  dst_v[r], dsem[r]).wait()
            pltpu.make_async_copy(x_hbm.at[src_slice(i)],
                                  rows_v[r], gsem[r]).wait()

        def wait_scatter(r):
            pltpu.make_async_copy(rows_v[r], acc_sh.at[dst_v[r]],
                                  ssem[r]).wait()

        def chunk_body(i, u, first=False, last=False):
            r = u % NR
            wait_chunk(i, r)
            def scale(g, c2):
                w16 = w_v[r][pl.ds(pl.multiple_of(g * L, L), L)]
                for e16 in range(L):
                    wsp = lax.gather(
                        w16, (zero16 + e16)[:, None], dimension_numbers=_GDN,
                        slice_sizes=(1,),
                        mode=lax.GatherScatterMode.PROMISE_IN_BOUNDS)
                    eg = g * L + e16
                    for c in range(D // L):
                        csl = pl.ds(c * L, L)
                        rows_v[r][eg, csl] = rows_v[r][eg, csl] * wsp
                return c2

            lax.fori_loop(0, K // L, scale, 0)
            pltpu.async_copy(rows_v[r], acc_sh.at[dst_v[r]],
                             ssem[r], add=True)
            if not first:
                wait_scatter((r + 2) % NR)
            if not last:
                start_chunk(i + 2, (r + 2) % NR)

        for u in range(2):
            start_chunk(u, u)
        for u in range(3):
            chunk_body(u, u, first=(u == 0))

        def group(p, carry):
            for u in range(3):
                chunk_body(3 * p + u, u)
            return carry

        lax.fori_loop(1, nck // 3 - 1, group, 0)
        for u in range(3):
            i = nck - 3 + u
            chunk_body(i, u, last=(u >= 1))
        wait_scatter(2)
        plsc.subcore_barrier()

        for j in range(ROWS_PER_TILE // K):
            row0 = sid * ROWS_PER_TILE + j * K
            pltpu.sync_copy(acc_sh.at[pl.ds(row0, K)], rows_v[0])
            pltpu.sync_copy(
                rows_v[0], out_hbm.at[pl.ds(cid * N_PAD + row0, K)])

    return body(x, src, dst, ew)


def _finish(partials, x, w1, w2, b):
    def body(p_ref, x_ref, w1_ref, w2_ref, b_ref, o_ref):
        g = p_ref[0] + p_ref[1]
        acc = jnp.dot(x_ref[...], w1_ref[...],
                      preferred_element_type=jnp.float32)
        acc += jnp.dot(g, w2_ref[...], preferred_element_type=jnp.float32)
        o_ref[...] = jnp.maximum(acc + b_ref[...], 0.0)

    grid = 10
    blk = N_NODES // grid
    return pl.pallas_call(
        body,
        grid=(grid,),
        in_specs=[
            pl.BlockSpec((NC, blk, D), lambda i: (0, i, 0)),
            pl.BlockSpec((blk, D), lambda i: (i, 0)),
            pl.BlockSpec((D, D), lambda i: (0, 0)),
            pl.BlockSpec((D, D), lambda i: (0, 0)),
            pl.BlockSpec((D,), lambda i: (0,)),
        ],
        out_specs=pl.BlockSpec((blk, D), lambda i: (i, 0)),
        out_shape=jax.ShapeDtypeStruct((N_NODES, D), jnp.float32),
    )(partials, x, w1, w2, b)


@jax.jit
def kernel(X, edge_index, edge_weight, w1, w2, b):
    src = edge_index[1].astype(jnp.int32)
    dst = edge_index[0].astype(jnp.int32)
    pad = E_PAD - N_EDGES
    src_p = jnp.concatenate([src, jnp.zeros((pad,), jnp.int32)])
    dst_p = jnp.concatenate([dst, jnp.zeros((pad,), jnp.int32)])
    ew_p = jnp.concatenate([edge_weight, jnp.zeros((pad,), jnp.float32)])
    p = _edge_agg(X, src_p, dst_p, ew_p)
    return _finish(p.reshape(NC, N_PAD, D), X, w1, w2, b)
